# baseline (speedup 1.0000x reference)
import functools

import jax
import jax.numpy as jnp
from jax import lax
from jax.experimental import pallas as pl
from jax.experimental.pallas import tpu as pltpu
from jax.experimental.pallas import tpu_sc as plsc

N = 20000
NPAD = 20480
T = 16
CH = NPAD // T
VR = CH // 16
FULL_T = N // CH
TAIL = N - FULL_T * CH
RADIX = 256
IR = 128
NR = CH // IR

_MESH = plsc.VectorSubcoreMesh(core_axis_name="c", subcore_axis_name="s")


@functools.partial(
    pl.kernel,
    out_type=jax.ShapeDtypeStruct((2 * 3 * NPAD,), jnp.float32),
    mesh=_MESH,
    compiler_params=pltpu.CompilerParams(needs_layout_passes=False),
    scratch_types=[
        pltpu.VMEM((CH * 16,), jnp.float32),
        pltpu.VMEM((NPAD,), jnp.int32),
        pltpu.VMEM((CH,), jnp.int32),
        pltpu.VMEM((3 * NR, IR), jnp.int32),
        pltpu.VMEM((RADIX,), jnp.int32),
        pltpu.VMEM((RADIX,), jnp.int32),
        pltpu.VMEM((T, RADIX), jnp.int32),
        pltpu.VMEM((3 * CH,), jnp.float32),
        pltpu.VMEM_SHARED((T, RADIX), jnp.int32),
        pltpu.VMEM_SHARED((NPAD,), jnp.int32),
        pltpu.VMEM_SHARED((NPAD,), jnp.int32),
        pltpu.VMEM_SHARED((NPAD,), jnp.int32),
        pltpu.SemaphoreType.DMA,
    ],
)
def _sc_sort(in0_hbm, in1_hbm, sorted_hbm, blk, m_all, iv, posb, hist, off,
             histall, rgbb, hist_sh, m_sh, iSh0, iSh1, sem):
    c = lax.axis_index("c")
    s = lax.axis_index("s")
    base = s * CH
    ones = jnp.ones((16,), jnp.int32)
    lane = lax.iota(jnp.int32, 16)

    @pl.when(jnp.logical_and(c == 0, s < FULL_T))
    def _():
        pltpu.sync_copy(in0_hbm.at[pl.ds(base * 16, CH * 16)], blk)

    @pl.when(jnp.logical_and(c == 1, s < FULL_T))
    def _():
        pltpu.sync_copy(in1_hbm.at[pl.ds(base * 16, CH * 16)], blk)

    @pl.when(jnp.logical_and(c == 0, s == FULL_T))
    def _():
        pltpu.sync_copy(in0_hbm.at[pl.ds(FULL_T * CH * 16, TAIL * 16)],
                        blk.at[pl.ds(0, TAIL * 16)])

    @pl.when(jnp.logical_and(c == 1, s == FULL_T))
    def _():
        pltpu.sync_copy(in1_hbm.at[pl.ds(FULL_T * CH * 16, TAIL * 16)],
                        blk.at[pl.ds(0, TAIL * 16)])

    def p0(v, _):
        idx = v * 16 + lane
        k = plsc.load_gather(blk, [idx * 16 + 1])
        b = plsc.bitcast(k + jnp.float32(0.0), jnp.int32)
        m = jnp.where(b >= 0, b ^ jnp.int32(-2**31), ~b)
        m = jnp.where(base + idx < N, m, jnp.int32(-1))
        m_all[pl.ds(v * 16, 16)] = m
        return 0

    lax.fori_loop(0, VR, p0, 0)
    pltpu.sync_copy(m_all.at[pl.ds(0, CH)], m_sh.at[pl.ds(base, CH)])
    plsc.subcore_barrier()
    pltpu.sync_copy(m_sh, m_all)

    def run_pass(p, i_src, i_dst):
        sh = 8 * p
        if i_src is None:
            def dig(v):
                return m_all[pl.ds(base + v * 16, 16)]
        else:
            pltpu.sync_copy(i_src.at[pl.ds(base, CH)], iv)

            def dig(v):
                return plsc.load_gather(m_all, [iv[pl.ds(v * 16, 16)]])

        def zero(j, _):
            hist[pl.ds(j * 16, 16)] = jnp.zeros((16,), jnp.int32)
            return 0
        lax.fori_loop(0, RADIX // 16, zero, 0)

        def histo(v, _):
            d = lax.shift_right_logical(dig(v), sh) & 255
            plsc.addupdate_scatter(hist, [d], ones)
            return 0
        lax.fori_loop(0, VR, histo, 0)

        pltpu.sync_copy(hist, hist_sh.at[s])
        plsc.subcore_barrier()

        pltpu.sync_copy(hist_sh, histall)

        def offs(j, carry):
            def acc(t, tp):
                tot, pri = tp
                h = histall[t, pl.ds(j * 16, 16)]
                return tot + h, pri + jnp.where(t < s, h, jnp.int32(0))
            tot, pri = lax.fori_loop(
                0, T, acc, (jnp.zeros((16,), jnp.int32),
                            jnp.zeros((16,), jnp.int32)))
            incl = plsc.cumsum(tot)
            off[pl.ds(j * 16, 16)] = carry + (incl - tot) + pri
            return carry + jnp.max(incl)
        lax.fori_loop(0, RADIX // 16, offs, jnp.int32(0))

        def posl(v, _):
            d = lax.shift_right_logical(dig(v), sh) & 255
            cur = plsc.load_gather(off, [d])
            occ, _unused = plsc.scan_count(d)
            posb[v // 8, pl.ds((v % 8) * 16, 16)] = cur + occ - 1
            plsc.addupdate_scatter(off, [d], ones)
            return 0
        lax.fori_loop(0, VR, posl, 0)

        if i_src is None:
            def cp(v, _):
                iv[pl.ds(v * 16, 16)] = base + v * 16 + lane
                return 0
            lax.fori_loop(0, VR, cp, 0)
        copies = [pltpu.async_copy(iv.at[pl.ds(j * IR, IR)],
                                   i_dst.at[posb.at[j]], sem)
                  for j in range(NR)]
        for cpy in copies:
            cpy.wait()
        plsc.subcore_barrier()

    run_pass(0, None, iSh0)
    run_pass(1, iSh0, iSh1)
    run_pass(2, iSh1, iSh0)
    run_pass(3, iSh0, iSh1)

    pltpu.sync_copy(iSh1.at[pl.ds(base, CH)], iv)

    def cpyi(v, _):
        e = jnp.minimum(iv[pl.ds(v * 16, 16)], jnp.int32(N - 1)) * 16
        for col in range(3):
            posb[col * NR + v // 8, pl.ds((v % 8) * 16, 16)] = e + (4 + col)
        return 0
    lax.fori_loop(0, VR, cpyi, 0)

    @pl.when(c == 0)
    def _():
        cps = [pltpu.async_copy(in0_hbm.at[posb.at[r]],
                                rgbb.at[pl.ds(r * IR, IR)], sem)
               for r in range(3 * NR)]
        for cpy in cps:
            cpy.wait()

    @pl.when(c == 1)
    def _():
        cps = [pltpu.async_copy(in1_hbm.at[posb.at[r]],
                                rgbb.at[pl.ds(r * IR, IR)], sem)
               for r in range(3 * NR)]
        for cpy in cps:
            cpy.wait()

    for col in range(3):
        pltpu.sync_copy(rgbb.at[pl.ds(col * CH, CH)],
                        sorted_hbm.at[pl.ds((c * 3 + col) * NPAD + base, CH)])


_CROWS = 3 * NPAD // 128


def _cmp_body(s_ref, o_ref):
    x = s_ref[...]
    f = (lax.broadcasted_iota(jnp.int32, (_CROWS, 128), 0) * 128
         + lax.broadcasted_iota(jnp.int32, (_CROWS, 128), 1))
    pos = f - (f // NPAD) * NPAD
    bad = jnp.where((x[:_CROWS] != x[_CROWS:]) & (pos < N), 1.0, 0.0)
    o_ref[0, 0] = jnp.where(jnp.sum(bad) == 0.0, 1.0, 0.0)


def kernel(ocm0, ocm1):
    srt = _sc_sort(ocm0.reshape(-1), ocm1.reshape(-1))
    res = pl.pallas_call(
        _cmp_body,
        out_shape=jax.ShapeDtypeStruct((1, 1), jnp.float32),
        out_specs=pl.BlockSpec(memory_space=pltpu.SMEM),
    )(srt.reshape(2 * _CROWS, 128))
    return res.reshape(())


# speedup vs baseline: 1.1843x; 1.1843x over previous
import functools

import jax
import jax.numpy as jnp
from jax import lax
from jax.experimental import pallas as pl
from jax.experimental.pallas import tpu as pltpu
from jax.experimental.pallas import tpu_sc as plsc

N = 20000
NPAD = 20480
T = 16
CH = NPAD // T
VR = CH // 16
FULL_T = N // CH
TAIL = N - FULL_T * CH
RADIX = 256
IR = 128
NR = CH // IR
SUB = 320

_MESH = plsc.VectorSubcoreMesh(core_axis_name="c", subcore_axis_name="s")


@functools.partial(
    pl.kernel,
    out_type=jax.ShapeDtypeStruct((2 * 3 * NPAD,), jnp.float32),
    mesh=_MESH,
    compiler_params=pltpu.CompilerParams(needs_layout_passes=False),
    scratch_types=[
        pltpu.VMEM((SUB, 16), jnp.float32),
        pltpu.VMEM((CH,), jnp.float32),
        pltpu.VMEM((NPAD,), jnp.int32),
        pltpu.VMEM((CH,), jnp.int32),
        pltpu.VMEM((3 * NR, IR), jnp.int32),
        pltpu.VMEM((RADIX,), jnp.int32),
        pltpu.VMEM((RADIX,), jnp.int32),
        pltpu.VMEM((T, RADIX), jnp.int32),
        pltpu.VMEM((3 * CH,), jnp.float32),
        pltpu.VMEM_SHARED((T, RADIX), jnp.int32),
        pltpu.VMEM_SHARED((NPAD,), jnp.int32),
        pltpu.VMEM_SHARED((NPAD,), jnp.int32),
        pltpu.VMEM_SHARED((NPAD,), jnp.int32),
        pltpu.VMEM_SHARED((3 * NPAD,), jnp.float32),
        pltpu.SemaphoreType.DMA,
    ],
)
def _sc_sort(in0_hbm, in1_hbm, sorted_hbm, blk, kb, m_all, iv, posb, hist,
             off, histall, rgbb, hist_sh, m_sh, iSh0, iSh1, rgb_sh, sem):
    c = lax.axis_index("c")
    s = lax.axis_index("s")
    base = s * CH
    ones = jnp.ones((16,), jnp.int32)
    lane = lax.iota(jnp.int32, 16)

    def _load_cols(src, sizes):
        for q, sz in enumerate(sizes):
            pltpu.sync_copy(src.at[pl.ds(base + q * SUB, sz)],
                            blk.at[pl.ds(0, sz)])

            def ext(v, _):
                idx = v * 16 + lane
                k = plsc.load_gather(blk, [idx, ones])
                kb[pl.ds(q * SUB + v * 16, 16)] = k
                for col in range(3):
                    val = plsc.load_gather(
                        blk, [idx, jnp.full((16,), 4 + col, jnp.int32)])
                    rgbb[pl.ds(col * CH + q * SUB + v * 16, 16)] = val
                return 0
            lax.fori_loop(0, sz // 16, ext, 0)

    _FULL = (SUB, SUB, SUB, SUB)
    _TAILS = (SUB, SUB, TAIL - 2 * SUB)

    @pl.when(jnp.logical_and(c == 0, s < FULL_T))
    def _():
        _load_cols(in0_hbm, _FULL)

    @pl.when(jnp.logical_and(c == 1, s < FULL_T))
    def _():
        _load_cols(in1_hbm, _FULL)

    @pl.when(jnp.logical_and(c == 0, s == FULL_T))
    def _():
        _load_cols(in0_hbm, _TAILS)

    @pl.when(jnp.logical_and(c == 1, s == FULL_T))
    def _():
        _load_cols(in1_hbm, _TAILS)

    def p0(v, _):
        idx = v * 16 + lane
        k = kb[pl.ds(v * 16, 16)]
        b = plsc.bitcast(k + jnp.float32(0.0), jnp.int32)
        m = jnp.where(b >= 0, b ^ jnp.int32(-2**31), ~b)
        m = jnp.where(base + idx < N, m, jnp.int32(-1))
        m_all[pl.ds(v * 16, 16)] = m
        return 0

    lax.fori_loop(0, VR, p0, 0)
    pltpu.sync_copy(m_all.at[pl.ds(0, CH)], m_sh.at[pl.ds(base, CH)])
    for col in range(3):
        pltpu.sync_copy(rgbb.at[pl.ds(col * CH, CH)],
                        rgb_sh.at[pl.ds(col * NPAD + base, CH)])
    plsc.subcore_barrier()
    pltpu.sync_copy(m_sh, m_all)

    def run_pass(p, i_src, i_dst):
        sh = 8 * p
        if i_src is None:
            def dig(v):
                return m_all[pl.ds(base + v * 16, 16)]
        else:
            pltpu.sync_copy(i_src.at[pl.ds(base, CH)], iv)

            def dig(v):
                return plsc.load_gather(m_all, [iv[pl.ds(v * 16, 16)]])

        def zero(j, _):
            hist[pl.ds(j * 16, 16)] = jnp.zeros((16,), jnp.int32)
            return 0
        lax.fori_loop(0, RADIX // 16, zero, 0)

        def histo(v, _):
            d = lax.shift_right_logical(dig(v), sh) & 255
            plsc.addupdate_scatter(hist, [d], ones)
            return 0
        lax.fori_loop(0, VR, histo, 0)

        pltpu.sync_copy(hist, hist_sh.at[s])
        plsc.subcore_barrier()

        pltpu.sync_copy(hist_sh, histall)

        def offs(j, carry):
            def acc(t, tp):
                tot, pri = tp
                h = histall[t, pl.ds(j * 16, 16)]
                return tot + h, pri + jnp.where(t < s, h, jnp.int32(0))
            tot, pri = lax.fori_loop(
                0, T, acc, (jnp.zeros((16,), jnp.int32),
                            jnp.zeros((16,), jnp.int32)))
            incl = plsc.cumsum(tot)
            off[pl.ds(j * 16, 16)] = carry + (incl - tot) + pri
            return carry + jnp.max(incl)
        lax.fori_loop(0, RADIX // 16, offs, jnp.int32(0))

        def posl(v, _):
            d = lax.shift_right_logical(dig(v), sh) & 255
            cur = plsc.load_gather(off, [d])
            occ, _unused = plsc.scan_count(d)
            posb[v // 8, pl.ds((v % 8) * 16, 16)] = cur + occ - 1
            plsc.addupdate_scatter(off, [d], ones)
            return 0
        lax.fori_loop(0, VR, posl, 0)

        if i_src is None:
            def cp(v, _):
                iv[pl.ds(v * 16, 16)] = base + v * 16 + lane
                return 0
            lax.fori_loop(0, VR, cp, 0)
        copies = [pltpu.async_copy(iv.at[pl.ds(j * IR, IR)],
                                   i_dst.at[posb.at[j]], sem)
                  for j in range(NR)]
        for cpy in copies:
            cpy.wait()
        plsc.subcore_barrier()

    run_pass(0, None, iSh0)
    run_pass(1, iSh0, iSh1)
    run_pass(2, iSh1, iSh0)
    run_pass(3, iSh0, iSh1)

    pltpu.sync_copy(iSh1.at[pl.ds(base, CH)], iv)

    def cpyi(v, _):
        e = iv[pl.ds(v * 16, 16)]
        for col in range(3):
            posb[col * NR + v // 8, pl.ds((v % 8) * 16, 16)] = \
                col * NPAD + e
        return 0
    lax.fori_loop(0, VR, cpyi, 0)

    cps = [pltpu.async_copy(rgb_sh.at[posb.at[r]],
                            rgbb.at[pl.ds(r * IR, IR)], sem)
           for r in range(3 * NR)]
    for cpy in cps:
        cpy.wait()

    for col in range(3):
        pltpu.sync_copy(rgbb.at[pl.ds(col * CH, CH)],
                        sorted_hbm.at[pl.ds((c * 3 + col) * NPAD + base, CH)])


_CROWS = 3 * NPAD // 128


def _cmp_body(s_ref, o_ref):
    x = s_ref[...]
    f = (lax.broadcasted_iota(jnp.int32, (_CROWS, 128), 0) * 128
         + lax.broadcasted_iota(jnp.int32, (_CROWS, 128), 1))
    pos = f - (f // NPAD) * NPAD
    bad = jnp.where((x[:_CROWS] != x[_CROWS:]) & (pos < N), 1.0, 0.0)
    o_ref[0, 0] = jnp.where(jnp.sum(bad) == 0.0, 1.0, 0.0)


def kernel(ocm0, ocm1):
    srt = _sc_sort(ocm0, ocm1)
    res = pl.pallas_call(
        _cmp_body,
        out_shape=jax.ShapeDtypeStruct((1, 1), jnp.float32),
        out_specs=pl.BlockSpec(memory_space=pltpu.SMEM),
    )(srt.reshape(2 * _CROWS, 128))
    return res.reshape(())
